# Initial kernel scaffold; baseline (speedup 1.0000x reference)
import jax
import jax.numpy as jnp
from jax.experimental import pallas as pl


def kernel(inputs, mask, edge_index, edge_values, W_in, b_in, fc0_gamma, fc0_beta, fc0_W, fc0_b, fc1_gamma, fc1_beta, fc1_W, fc1_b, conv2_gamma, conv2_beta, conv2_W, conv2_b):
    raise NotImplementedError("write your pallas kernel here")


# speedup vs baseline: 2.8603x; 2.8603x over previous
import functools

import jax
import jax.numpy as jnp
from jax import lax
from jax.experimental import pallas as pl
from jax.experimental.pallas import tpu as pltpu
from jax.experimental.pallas import tpu_sc as plsc

_N = 10000
_E = 320000
_D = 128

_NC = 2
_NS = 16
_NW = _NC * _NS
_EPT = _E // _NW
_CH = 80
_NCHUNK = _EPT // _CH
_RC = 80
_NRC = _N // _RC
_RC_PER_S = -(-_NRC // _NS)


def _lap_sc_body(src_hbm, dst_hbm, vexp_hbm, y_hbm, out_hbm,
                 acc, srcb, dstb, vb, rowsb, sem):
    c = lax.axis_index("c")
    s = lax.axis_index("s")
    wid = s * _NC + c
    base = wid * _EPT

    def _zrow(i, carry):
        for f in range(_D // 16):
            rowsb[i, pl.ds(16 * f, 16)] = jnp.zeros((16,), jnp.float32)
        return carry
    lax.fori_loop(0, _RC, _zrow, 0)
    for j in range(_RC_PER_S):
        cid = s + _NS * j

        @pl.when(cid < _NRC)
        def _():
            pltpu.sync_copy(rowsb, acc.at[pl.ds(cid * _RC, _RC)])
    plsc.subcore_barrier()

    def _chunk(g, carry):
        off = base + g * _CH
        pltpu.sync_copy(src_hbm.at[pl.ds(off, _CH)], srcb)
        pltpu.sync_copy(dst_hbm.at[pl.ds(off, _CH)], dstb)
        pltpu.sync_copy(vexp_hbm.at[pl.ds(off, _CH)], vb)
        pltpu.async_copy(y_hbm.at[srcb], rowsb, sem).wait()

        def _edge(e, carry2):
            vv = vb[e, :]
            for f in range(_D // 16):
                rowsb[e, pl.ds(16 * f, 16)] = rowsb[e, pl.ds(16 * f, 16)] * vv
            return carry2
        lax.fori_loop(0, _CH, _edge, 0)
        pltpu.sync_copy(rowsb, acc.at[dstb], add=True)
        return carry
    lax.fori_loop(0, _NCHUNK, _chunk, 0)
    plsc.subcore_barrier()

    for j in range(_RC_PER_S):
        cid = s + _NS * j

        @pl.when(cid < _NRC)
        def _():
            r0 = cid * _RC
            pltpu.sync_copy(acc.at[pl.ds(r0, _RC)], rowsb)
            pltpu.sync_copy(rowsb, out_hbm.at[c, pl.ds(r0, _RC)])


@jax.jit
def _lap_sc(src, dst, vexp, y):
    mesh = plsc.VectorSubcoreMesh(core_axis_name="c", subcore_axis_name="s")
    f = pl.kernel(
        _lap_sc_body,
        out_type=jax.ShapeDtypeStruct((_NC, _N, _D), jnp.float32),
        mesh=mesh,
        scratch_types=[
            pltpu.VMEM_SHARED((_N, _D), jnp.float32),
            pltpu.VMEM((_CH,), jnp.int32),
            pltpu.VMEM((_CH,), jnp.int32),
            pltpu.VMEM((_CH, 16), jnp.float32),
            pltpu.VMEM((_CH, _D), jnp.float32),
            pltpu.SemaphoreType.DMA,
        ],
    )
    return f(src, dst, vexp, y)


def _elu(x):
    return jnp.where(x > 0, x, jnp.exp(jnp.minimum(x, 0.0)) - 1.0)


def _bnmm(z, g, bt, w, b):
    mu = jnp.mean(z, axis=0, keepdims=True)
    var = jnp.mean((z - mu) * (z - mu), axis=0, keepdims=True)
    zh = (z - mu) * lax.rsqrt(var + 1e-5) * g + bt
    return jnp.dot(zh, w, preferred_element_type=jnp.float32) + b


def _in_body(inp_ref, w_ref, b_ref, x0_ref, a_ref):
    inp = inp_ref[...]
    w = w_ref[...]
    x0 = (inp[:, 0:1] * w[0:1, :] + inp[:, 1:2] * w[1:2, :]
          + inp[:, 2:3] * w[2:3, :]) + b_ref[...]
    x0_ref[...] = x0
    a_ref[...] = _elu(x0)


def _lapmid_body(y_ref, p_ref, g_ref, bt_ref, w_ref, b_ref, out_ref):
    y = y_ref[...]
    op = p_ref[0] + p_ref[1]
    z = jnp.concatenate([y, op], axis=1)
    out_ref[...] = _elu(_bnmm(z, g_ref[...], bt_ref[...], w_ref[...],
                              b_ref[...]))


def _lapend_body(y_ref, p_ref, res_ref, g_ref, bt_ref, w_ref, b_ref,
                 x_ref, a_ref):
    y = y_ref[...]
    op = p_ref[0] + p_ref[1]
    z = jnp.concatenate([y, op], axis=1)
    x = _bnmm(z, g_ref[...], bt_ref[...], w_ref[...], b_ref[...]) + res_ref[...]
    x_ref[...] = x
    a_ref[...] = _elu(x)


def _avg_layer_body(x_ref, a_ref, m_ref,
                    g0_ref, bt0_ref, w0_ref, b0_ref,
                    g1_ref, bt1_ref, w1_ref, b1_ref,
                    x2_ref, a2_ref):
    xin = x_ref[...]
    a = a_ref[...]
    m = m_ref[...]
    den = jnp.sum(m)
    s = jnp.sum(a * m, axis=0, keepdims=True) / den
    z = jnp.concatenate([a, jnp.broadcast_to(s, a.shape)], axis=1)
    u = _elu(_bnmm(z, g0_ref[...], bt0_ref[...], w0_ref[...], b0_ref[...]))
    s2 = jnp.sum(u * m, axis=0, keepdims=True) / den
    z2 = jnp.concatenate([u, jnp.broadcast_to(s2, u.shape)], axis=1)
    x2 = _bnmm(z2, g1_ref[...], bt1_ref[...], w1_ref[...], b1_ref[...]) + xin
    x2_ref[...] = x2
    a2_ref[...] = _elu(x2)


def _head_body(a_ref, t_ref, g_ref, bt_ref, w_ref, b_ref, out_ref):
    out_ref[...] = _bnmm(a_ref[...], g_ref[...], bt_ref[...], w_ref[...],
                         b_ref[...]) + t_ref[...]


def _tc(body, out_shapes):
    return pl.pallas_call(
        body,
        out_shape=out_shapes,
    )


_F32 = functools.partial(jax.ShapeDtypeStruct, dtype=jnp.float32)


def kernel(inputs, mask, edge_index, edge_values, W_in, b_in,
           fc0_gamma, fc0_beta, fc0_W, fc0_b,
           fc1_gamma, fc1_beta, fc1_W, fc1_b,
           conv2_gamma, conv2_beta, conv2_W, conv2_b):
    inp = inputs[0]
    m = mask[0]
    src = edge_index[0]
    dst = edge_index[1]
    vexp = jnp.broadcast_to(edge_values[:, None], (_E, 16))

    x0, a0 = _tc(_in_body, [_F32((_N, _D)), _F32((_N, _D))])(
        inp, W_in, b_in[None, :])

    def lap(y):
        return _lap_sc(src, dst, vexp, y)

    def lapmid(y, p, i):
        return _tc(_lapmid_body, _F32((_N, _D)))(
            y, p, fc0_gamma[i][None, :], fc0_beta[i][None, :], fc0_W[i],
            fc0_b[i][None, :])

    def lapend(y, p, res, i):
        return _tc(_lapend_body, [_F32((_N, _D)), _F32((_N, _D))])(
            y, p, res, fc1_gamma[i][None, :], fc1_beta[i][None, :], fc1_W[i],
            fc1_b[i][None, :])

    def avg_layer(x, a, i):
        return _tc(_avg_layer_body, [_F32((_N, _D)), _F32((_N, _D))])(
            x, a, m,
            fc0_gamma[i][None, :], fc0_beta[i][None, :], fc0_W[i],
            fc0_b[i][None, :],
            fc1_gamma[i][None, :], fc1_beta[i][None, :], fc1_W[i],
            fc1_b[i][None, :])

    a1 = lapmid(a0, lap(a0), 0)
    x1, a2 = lapend(a1, lap(a1), x0, 0)
    x2, a3 = avg_layer(x1, a2, 1)
    a4 = lapmid(a3, lap(a3), 2)
    x3, a5 = lapend(a4, lap(a4), x2, 2)
    x4, a6 = avg_layer(x3, a5, 3)

    tiled = jnp.broadcast_to(inp[:, None, :], (_N, 40, 3)).reshape(_N, 120)
    out = _tc(_head_body, _F32((_N, 120)))(
        a6, tiled, conv2_gamma[None, :], conv2_beta[None, :], conv2_W,
        conv2_b[None, :])
    return out[None]


# speedup vs baseline: 3.2241x; 1.1272x over previous
import functools

import jax
import jax.numpy as jnp
from jax import lax
from jax.experimental import pallas as pl
from jax.experimental.pallas import tpu as pltpu
from jax.experimental.pallas import tpu_sc as plsc

_N = 10000
_E = 320000
_D = 128

_NC = 2
_NS = 16
_NW = _NC * _NS
_EPT = _E // _NW
_CH = 128
_CPW = 80
_NPH = 2
_CPP = _CPW // _NPH
_EPAD = _NW * _CPW * _CH
_RC = 80
_NRC = _N // _RC
_RC_PER_S = -(-_NRC // _NS)
del _EPT


_GATHER_DNUMS = lax.GatherDimensionNumbers(
    offset_dims=(), collapsed_slice_dims=(0,), start_index_map=(0,))


def _bcast_lane(v16, e):
    idx = jnp.full((16, 1), e, jnp.int32)
    return lax.gather(v16, idx, _GATHER_DNUMS, (1,),
                      mode=lax.GatherScatterMode.PROMISE_IN_BOUNDS)


def _lap_sc_body(src_hbm, dst_hbm, vals_hbm, y_hbm, out_hbm,
                 acc, srcb, dstb, valsb, rows0, rows1, gs0, gs1):
    c = lax.axis_index("c")
    s = lax.axis_index("s")
    wid = s * _NC + c

    def _zrow(i, carry):
        for f in range(_D // 16):
            rows0[i, pl.ds(16 * f, 16)] = jnp.zeros((16,), jnp.float32)
        return carry
    lax.fori_loop(0, _RC, _zrow, 0)
    zsrc = rows0.at[pl.ds(0, _RC)]
    for j in range(_RC_PER_S):
        cid = s + _NS * j

        @pl.when(cid < _NRC)
        def _():
            pltpu.sync_copy(zsrc, acc.at[pl.ds(cid * _RC, _RC)])
    plsc.subcore_barrier()

    def _gather_start(l, rows, sem):
        pltpu.async_copy(y_hbm.at[srcb.at[l]], rows, sem)

    def _gather_wait(rows, sem):
        pltpu.make_async_copy(y_hbm.at[pl.ds(0, _CH)], rows, sem).wait()

    def _scale(l, rows):
        def _grp(jj, carry):
            off = pl.multiple_of(jj * 16, 16)
            vv16 = valsb[l, pl.ds(off, 16)]
            for ee in range(16):
                e = jj * 16 + ee
                sc = _bcast_lane(vv16, ee)
                for f in range(_D // 16):
                    rows[e, pl.ds(16 * f, 16)] = (
                        rows[e, pl.ds(16 * f, 16)] * sc)
            return carry
        lax.fori_loop(0, _CH // 16, _grp, 0)

    def _scatter(l, rows):
        pltpu.sync_copy(rows, acc.at[dstb.at[l]], add=True)

    for ph in range(_NPH):
        row0 = wid * _CPW + ph * _CPP
        pltpu.sync_copy(src_hbm.at[pl.ds(row0, _CPP)], srcb)
        pltpu.sync_copy(dst_hbm.at[pl.ds(row0, _CPP)], dstb)
        pltpu.sync_copy(vals_hbm.at[pl.ds(row0, _CPP)], valsb)
        _gather_start(0, rows0, gs0)

        def _pair(i, carry):
            c0 = 2 * i
            _gather_start(c0 + 1, rows1, gs1)
            _gather_wait(rows0, gs0)
            _scale(c0, rows0)
            _scatter(c0, rows0)

            @pl.when(c0 + 2 < _CPP)
            def _():
                _gather_start(c0 + 2, rows0, gs0)
            _gather_wait(rows1, gs1)
            _scale(c0 + 1, rows1)
            _scatter(c0 + 1, rows1)
            return carry
        lax.fori_loop(0, _CPP // 2, _pair, 0)
    plsc.subcore_barrier()

    for j in range(_RC_PER_S):
        cid = s + _NS * j

        @pl.when(cid < _NRC)
        def _():
            r0 = cid * _RC
            bounce = rows0.at[pl.ds(0, _RC)]
            pltpu.sync_copy(acc.at[pl.ds(r0, _RC)], bounce)
            pltpu.sync_copy(bounce, out_hbm.at[c, pl.ds(r0, _RC)])


@jax.jit
def _lap_sc(src2, dst2, vals2, y):
    mesh = plsc.VectorSubcoreMesh(core_axis_name="c", subcore_axis_name="s")
    f = pl.kernel(
        _lap_sc_body,
        out_type=jax.ShapeDtypeStruct((_NC, _N, _D), jnp.float32),
        mesh=mesh,
        scratch_types=[
            pltpu.VMEM_SHARED((_N, _D), jnp.float32),
            pltpu.VMEM((_CPP, _CH), jnp.int32),
            pltpu.VMEM((_CPP, _CH), jnp.int32),
            pltpu.VMEM((_CPP, _CH), jnp.float32),
            pltpu.VMEM((_CH, _D), jnp.float32),
            pltpu.VMEM((_CH, _D), jnp.float32),
            pltpu.SemaphoreType.DMA,
            pltpu.SemaphoreType.DMA,
        ],
    )
    return f(src2, dst2, vals2, y)


def _elu(x):
    return jnp.where(x > 0, x, jnp.exp(jnp.minimum(x, 0.0)) - 1.0)


def _bnmm(z, g, bt, w, b):
    mu = jnp.mean(z, axis=0, keepdims=True)
    var = jnp.mean((z - mu) * (z - mu), axis=0, keepdims=True)
    zh = (z - mu) * lax.rsqrt(var + 1e-5) * g + bt
    return jnp.dot(zh, w, preferred_element_type=jnp.float32) + b


def _in_body(inp_ref, w_ref, b_ref, x0_ref, a_ref):
    inp = inp_ref[...]
    w = w_ref[...]
    x0 = (inp[:, 0:1] * w[0:1, :] + inp[:, 1:2] * w[1:2, :]
          + inp[:, 2:3] * w[2:3, :]) + b_ref[...]
    x0_ref[...] = x0
    a_ref[...] = _elu(x0)


def _lapmid_body(y_ref, p_ref, g_ref, bt_ref, w_ref, b_ref, out_ref):
    y = y_ref[...]
    op = p_ref[0] + p_ref[1]
    z = jnp.concatenate([y, op], axis=1)
    out_ref[...] = _elu(_bnmm(z, g_ref[...], bt_ref[...], w_ref[...],
                              b_ref[...]))


def _lapend_body(y_ref, p_ref, res_ref, g_ref, bt_ref, w_ref, b_ref,
                 x_ref, a_ref):
    y = y_ref[...]
    op = p_ref[0] + p_ref[1]
    z = jnp.concatenate([y, op], axis=1)
    x = _bnmm(z, g_ref[...], bt_ref[...], w_ref[...], b_ref[...]) + res_ref[...]
    x_ref[...] = x
    a_ref[...] = _elu(x)


def _avg_layer_body(x_ref, a_ref, m_ref,
                    g0_ref, bt0_ref, w0_ref, b0_ref,
                    g1_ref, bt1_ref, w1_ref, b1_ref,
                    x2_ref, a2_ref):
    xin = x_ref[...]
    a = a_ref[...]
    m = m_ref[...]
    den = jnp.sum(m)
    s = jnp.sum(a * m, axis=0, keepdims=True) / den
    z = jnp.concatenate([a, jnp.broadcast_to(s, a.shape)], axis=1)
    u = _elu(_bnmm(z, g0_ref[...], bt0_ref[...], w0_ref[...], b0_ref[...]))
    s2 = jnp.sum(u * m, axis=0, keepdims=True) / den
    z2 = jnp.concatenate([u, jnp.broadcast_to(s2, u.shape)], axis=1)
    x2 = _bnmm(z2, g1_ref[...], bt1_ref[...], w1_ref[...], b1_ref[...]) + xin
    x2_ref[...] = x2
    a2_ref[...] = _elu(x2)


def _head_body(a_ref, t_ref, g_ref, bt_ref, w_ref, b_ref, out_ref):
    out_ref[...] = _bnmm(a_ref[...], g_ref[...], bt_ref[...], w_ref[...],
                         b_ref[...]) + t_ref[...]


def _tc(body, out_shapes):
    return pl.pallas_call(
        body,
        out_shape=out_shapes,
    )


_F32 = functools.partial(jax.ShapeDtypeStruct, dtype=jnp.float32)


def kernel(inputs, mask, edge_index, edge_values, W_in, b_in,
           fc0_gamma, fc0_beta, fc0_W, fc0_b,
           fc1_gamma, fc1_beta, fc1_W, fc1_b,
           conv2_gamma, conv2_beta, conv2_W, conv2_b):
    inp = inputs[0]
    m = mask[0]
    pad = _EPAD - _E
    src2 = jnp.concatenate(
        [edge_index[0], jnp.zeros((pad,), edge_index.dtype)]).reshape(
            _EPAD // _CH, _CH)
    dst2 = jnp.concatenate(
        [edge_index[1], jnp.zeros((pad,), edge_index.dtype)]).reshape(
            _EPAD // _CH, _CH)
    vals2 = jnp.concatenate(
        [edge_values, jnp.zeros((pad,), edge_values.dtype)]).reshape(
            _EPAD // _CH, _CH)

    x0, a0 = _tc(_in_body, [_F32((_N, _D)), _F32((_N, _D))])(
        inp, W_in, b_in[None, :])

    def lap(y):
        return _lap_sc(src2, dst2, vals2, y)

    def lapmid(y, p, i):
        return _tc(_lapmid_body, _F32((_N, _D)))(
            y, p, fc0_gamma[i][None, :], fc0_beta[i][None, :], fc0_W[i],
            fc0_b[i][None, :])

    def lapend(y, p, res, i):
        return _tc(_lapend_body, [_F32((_N, _D)), _F32((_N, _D))])(
            y, p, res, fc1_gamma[i][None, :], fc1_beta[i][None, :], fc1_W[i],
            fc1_b[i][None, :])

    def avg_layer(x, a, i):
        return _tc(_avg_layer_body, [_F32((_N, _D)), _F32((_N, _D))])(
            x, a, m,
            fc0_gamma[i][None, :], fc0_beta[i][None, :], fc0_W[i],
            fc0_b[i][None, :],
            fc1_gamma[i][None, :], fc1_beta[i][None, :], fc1_W[i],
            fc1_b[i][None, :])

    a1 = lapmid(a0, lap(a0), 0)
    x1, a2 = lapend(a1, lap(a1), x0, 0)
    x2, a3 = avg_layer(x1, a2, 1)
    a4 = lapmid(a3, lap(a3), 2)
    x3, a5 = lapend(a4, lap(a4), x2, 2)
    x4, a6 = avg_layer(x3, a5, 3)

    tiled = jnp.broadcast_to(inp[:, None, :], (_N, 40, 3)).reshape(_N, 120)
    out = _tc(_head_body, _F32((_N, 120)))(
        a6, tiled, conv2_gamma[None, :], conv2_beta[None, :], conv2_W,
        conv2_b[None, :])
    return out[None]


# speedup vs baseline: 9.0608x; 2.8103x over previous
import functools

import jax
import jax.numpy as jnp
from jax import lax
from jax.experimental import pallas as pl
from jax.experimental.pallas import tpu as pltpu
from jax.experimental.pallas import tpu_sc as plsc

_N = 10000
_E = 320000
_D = 128

_NC = 2
_NS = 16
_NW = _NC * _NS
_EPT = _E // _NW
_CH = 128
_CPW = 80
_NPH = 2
_CPP = _CPW // _NPH
_EPAD = _NW * _CPW * _CH
_RC = 80
_NRC = _N // _RC
_RC_PER_S = -(-_NRC // _NS)
del _EPT


_GATHER_DNUMS = lax.GatherDimensionNumbers(
    offset_dims=(), collapsed_slice_dims=(0,), start_index_map=(0,))


def _bcast_lane(v16, e):
    idx = jnp.full((16, 1), e, jnp.int32)
    return lax.gather(v16, idx, _GATHER_DNUMS, (1,),
                      mode=lax.GatherScatterMode.PROMISE_IN_BOUNDS)


def _lap_sc_body(src_hbm, dst_hbm, vals_hbm, y_hbm, out_hbm,
                 acc, srcb, dstb, valsb, rows0, rows1, gs0, gs1):
    c = lax.axis_index("c")
    s = lax.axis_index("s")
    wid = s * _NC + c

    def _zrow(i, carry):
        for f in range(_D // 16):
            rows0[i, pl.ds(16 * f, 16)] = jnp.zeros((16,), jnp.float32)
        return carry
    lax.fori_loop(0, _RC, _zrow, 0)
    zsrc = rows0.at[pl.ds(0, _RC)]
    for j in range(_RC_PER_S):
        cid = s + _NS * j

        @pl.when(cid < _NRC)
        def _():
            pltpu.sync_copy(zsrc, acc.at[pl.ds(cid * _RC, _RC)])
    plsc.subcore_barrier()

    def _gather_start(l, rows, sem):
        pltpu.async_copy(y_hbm.at[srcb.at[l]], rows, sem)

    def _gather_wait(rows, sem):
        pltpu.make_async_copy(y_hbm.at[pl.ds(0, _CH)], rows, sem).wait()

    def _scale(l, rows):
        def _grp(jj, carry):
            off = pl.multiple_of(jj * 16, 16)
            vv16 = valsb[l, pl.ds(off, 16)]
            for ee in range(16):
                e = jj * 16 + ee
                sc = _bcast_lane(vv16, ee)
                for f in range(_D // 16):
                    rows[e, pl.ds(16 * f, 16)] = (
                        rows[e, pl.ds(16 * f, 16)] * sc)
            return carry
        lax.fori_loop(0, _CH // 16, _grp, 0)

    def _scatter(l, rows):
        pltpu.sync_copy(rows, acc.at[dstb.at[l]], add=True)

    for ph in range(_NPH):
        row0 = wid * _CPW + ph * _CPP
        pltpu.sync_copy(src_hbm.at[pl.ds(row0, _CPP)], srcb)
        pltpu.sync_copy(dst_hbm.at[pl.ds(row0, _CPP)], dstb)
        pltpu.sync_copy(vals_hbm.at[pl.ds(row0, _CPP)], valsb)
        _gather_start(0, rows0, gs0)

        def _pair(i, carry):
            c0 = 2 * i
            _gather_start(c0 + 1, rows1, gs1)
            _gather_wait(rows0, gs0)
            _scale(c0, rows0)
            _scatter(c0, rows0)

            @pl.when(c0 + 2 < _CPP)
            def _():
                _gather_start(c0 + 2, rows0, gs0)
            _gather_wait(rows1, gs1)
            _scale(c0 + 1, rows1)
            _scatter(c0 + 1, rows1)
            return carry
        lax.fori_loop(0, _CPP // 2, _pair, 0)
    plsc.subcore_barrier()

    for j in range(_RC_PER_S):
        cid = s + _NS * j

        @pl.when(cid < _NRC)
        def _():
            r0 = cid * _RC
            bounce = rows0.at[pl.ds(0, _RC)]
            pltpu.sync_copy(acc.at[pl.ds(r0, _RC)], bounce)
            pltpu.sync_copy(bounce, out_hbm.at[c, pl.ds(r0, _RC)])


@jax.jit
def _lap_sc(src2, dst2, vals2, y):
    mesh = plsc.VectorSubcoreMesh(core_axis_name="c", subcore_axis_name="s")
    f = pl.kernel(
        _lap_sc_body,
        out_type=jax.ShapeDtypeStruct((_NC, _N, _D), jnp.float32),
        mesh=mesh,
        scratch_types=[
            pltpu.VMEM_SHARED((_N, _D), jnp.float32),
            pltpu.VMEM((_CPP, _CH), jnp.int32),
            pltpu.VMEM((_CPP, _CH), jnp.int32),
            pltpu.VMEM((_CPP, _CH), jnp.float32),
            pltpu.VMEM((_CH, _D), jnp.float32),
            pltpu.VMEM((_CH, _D), jnp.float32),
            pltpu.SemaphoreType.DMA,
            pltpu.SemaphoreType.DMA,
        ],
    )
    return f(src2, dst2, vals2, y)


def _elu(x):
    return jnp.where(x > 0, x, jnp.exp(jnp.minimum(x, 0.0)) - 1.0)


def _bnmm(z, g, bt, w, b):
    mu = jnp.mean(z, axis=0, keepdims=True)
    var = jnp.mean((z - mu) * (z - mu), axis=0, keepdims=True)
    zh = (z - mu) * lax.rsqrt(var + 1e-5) * g + bt
    return jnp.dot(zh, w, preferred_element_type=jnp.float32) + b


def _in_body(inp_ref, w_ref, b_ref, x0_ref, a_ref):
    inp = inp_ref[...]
    w = w_ref[...]
    x0 = (inp[:, 0:1] * w[0:1, :] + inp[:, 1:2] * w[1:2, :]
          + inp[:, 2:3] * w[2:3, :]) + b_ref[...]
    x0_ref[...] = x0
    a_ref[...] = _elu(x0)


def _lapmid_body(y_ref, p_ref, g_ref, bt_ref, w_ref, b_ref, out_ref):
    y = y_ref[...]
    op = p_ref[0] + p_ref[1]
    z = jnp.concatenate([y, op], axis=1)
    out_ref[...] = _elu(_bnmm(z, g_ref[...], bt_ref[...], w_ref[...],
                              b_ref[...]))


def _lapend_body(y_ref, p_ref, res_ref, g_ref, bt_ref, w_ref, b_ref,
                 x_ref, a_ref):
    y = y_ref[...]
    op = p_ref[0] + p_ref[1]
    z = jnp.concatenate([y, op], axis=1)
    x = _bnmm(z, g_ref[...], bt_ref[...], w_ref[...], b_ref[...]) + res_ref[...]
    x_ref[...] = x
    a_ref[...] = _elu(x)


def _avg_layer_body(x_ref, a_ref, m_ref,
                    g0_ref, bt0_ref, w0_ref, b0_ref,
                    g1_ref, bt1_ref, w1_ref, b1_ref,
                    x2_ref, a2_ref):
    xin = x_ref[...]
    a = a_ref[...]
    m = m_ref[...]
    den = jnp.sum(m)
    s = jnp.sum(a * m, axis=0, keepdims=True) / den
    z = jnp.concatenate([a, jnp.broadcast_to(s, a.shape)], axis=1)
    u = _elu(_bnmm(z, g0_ref[...], bt0_ref[...], w0_ref[...], b0_ref[...]))
    s2 = jnp.sum(u * m, axis=0, keepdims=True) / den
    z2 = jnp.concatenate([u, jnp.broadcast_to(s2, u.shape)], axis=1)
    x2 = _bnmm(z2, g1_ref[...], bt1_ref[...], w1_ref[...], b1_ref[...]) + xin
    x2_ref[...] = x2
    a2_ref[...] = _elu(x2)


def _head_body(a_ref, t_ref, g_ref, bt_ref, w_ref, b_ref, out_ref):
    out_ref[...] = _bnmm(a_ref[...], g_ref[...], bt_ref[...], w_ref[...],
                         b_ref[...]) + t_ref[...]


def _tc(body, out_shapes):
    return pl.pallas_call(
        body,
        out_shape=out_shapes,
    )


_F32 = functools.partial(jax.ShapeDtypeStruct, dtype=jnp.float32)


def kernel(inputs, mask, edge_index, edge_values, W_in, b_in,
           fc0_gamma, fc0_beta, fc0_W, fc0_b,
           fc1_gamma, fc1_beta, fc1_W, fc1_b,
           conv2_gamma, conv2_beta, conv2_W, conv2_b):
    inp = inputs[0]
    m = mask[0]
    pad = _EPAD - _E
    spread = (jnp.arange(pad, dtype=edge_index.dtype) * 37) % _N
    src2 = jnp.concatenate([edge_index[0], spread]).reshape(
        _EPAD // _CH, _CH)
    dst2 = jnp.concatenate([edge_index[1], spread]).reshape(
        _EPAD // _CH, _CH)
    vals2 = jnp.concatenate(
        [edge_values, jnp.zeros((pad,), edge_values.dtype)]).reshape(
            _EPAD // _CH, _CH)

    x0, a0 = _tc(_in_body, [_F32((_N, _D)), _F32((_N, _D))])(
        inp, W_in, b_in[None, :])

    def lap(y):
        return _lap_sc(src2, dst2, vals2, y)

    def lapmid(y, p, i):
        return _tc(_lapmid_body, _F32((_N, _D)))(
            y, p, fc0_gamma[i][None, :], fc0_beta[i][None, :], fc0_W[i],
            fc0_b[i][None, :])

    def lapend(y, p, res, i):
        return _tc(_lapend_body, [_F32((_N, _D)), _F32((_N, _D))])(
            y, p, res, fc1_gamma[i][None, :], fc1_beta[i][None, :], fc1_W[i],
            fc1_b[i][None, :])

    def avg_layer(x, a, i):
        return _tc(_avg_layer_body, [_F32((_N, _D)), _F32((_N, _D))])(
            x, a, m,
            fc0_gamma[i][None, :], fc0_beta[i][None, :], fc0_W[i],
            fc0_b[i][None, :],
            fc1_gamma[i][None, :], fc1_beta[i][None, :], fc1_W[i],
            fc1_b[i][None, :])

    a1 = lapmid(a0, lap(a0), 0)
    x1, a2 = lapend(a1, lap(a1), x0, 0)
    x2, a3 = avg_layer(x1, a2, 1)
    a4 = lapmid(a3, lap(a3), 2)
    x3, a5 = lapend(a4, lap(a4), x2, 2)
    x4, a6 = avg_layer(x3, a5, 3)

    tiled = jnp.broadcast_to(inp[:, None, :], (_N, 40, 3)).reshape(_N, 120)
    out = _tc(_head_body, _F32((_N, 120)))(
        a6, tiled, conv2_gamma[None, :], conv2_beta[None, :], conv2_W,
        conv2_b[None, :])
    return out[None]
